# baseline (speedup 1.0000x reference)
import functools

import jax
import jax.numpy as jnp
from jax import lax
from jax.experimental import pallas as pl
from jax.experimental.pallas import tpu as pltpu
from jax.experimental.pallas import tpu_sc as plsc

N = 10000
NP = 10240
F = 128
E = 320000
O = 40

NC = 2
NS = 16
RPT = NP // NS
RC = 64
EDGES_PER_SC = E // NC
EDGES_PER_TILE = EDGES_PER_SC // NS
SE = 128
EP = 327680
SAC = 10000
EPT = EP // (NC * NS)
NCHUNK = EPT // SE
K = 8
NGROUP = NCHUNK // K
EROWS = EP // SE

CH = 1000
NBLK = N // CH

_MESH = plsc.VectorSubcoreMesh(core_axis_name="c", subcore_axis_name="s")


def _init_acc(c, r0, pos_hbm, zeros_hbm, acc_sh):
    @pl.when(c == 0)
    def _():
        pltpu.sync_copy(pos_hbm.at[pl.ds(r0, RPT)], acc_sh.at[pl.ds(r0, RPT)])

    @pl.when(c != 0)
    def _():
        pltpu.sync_copy(zeros_hbm.at[pl.ds(r0, RPT)], acc_sh.at[pl.ds(r0, RPT)])


def _write_out(c, r0, out_hbm, acc_sh):
    pltpu.sync_copy(acc_sh.at[pl.ds(r0, RPT)], out_hbm.at[c, pl.ds(r0, RPT)])


@functools.partial(
    pl.kernel,
    out_type=jax.ShapeDtypeStruct((NC, NP, F), jnp.float32),
    mesh=_MESH,
    scratch_types=[
        pltpu.VMEM((NCHUNK, SE), jnp.int32),
        pltpu.VMEM((SE, F), jnp.float32),
        pltpu.VMEM_SHARED((NP, F), jnp.float32),
        pltpu.SemaphoreType.DMA((4,)),
    ],
)
def _deg_kernel(dst2_hbm, ones_hbm, zeros_hbm, out_hbm,
                di_all, ones_v, acc_sh, sem_s):
    c = lax.axis_index("c")
    s = lax.axis_index("s")
    r0 = s * RPT
    _init_acc(c, r0, ones_hbm, zeros_hbm, acc_sh)
    pltpu.sync_copy(ones_hbm.at[pl.ds(0, SE)], ones_v)
    row0 = (c * NS + s) * NCHUNK
    pltpu.sync_copy(dst2_hbm.at[pl.ds(row0, NCHUNK)], di_all)
    plsc.subcore_barrier()

    def group(j0, carry):
        base = j0 * 4
        descs = [
            pltpu.async_copy(ones_v, acc_sh.at[di_all.at[base + b]],
                             sem_s.at[b], add=True)
            for b in range(4)
        ]
        for d in descs:
            d.wait()
        return carry

    lax.fori_loop(0, NCHUNK // 4, group, 0)
    plsc.subcore_barrier()
    _write_out(c, r0, out_hbm, acc_sh)


SE2 = 64
NCH2 = EPT // SE2
NPAIR = NCH2 // 2


@functools.partial(
    pl.kernel,
    out_type=jax.ShapeDtypeStruct((NC, NP, F), jnp.float32),
    mesh=_MESH,
    scratch_types=[
        pltpu.VMEM((SE2,), jnp.int32),
        pltpu.VMEM((SE2,), jnp.int32),
        pltpu.VMEM((SE2,), jnp.int32),
        pltpu.VMEM((SE2,), jnp.int32),
        pltpu.VMEM((SE2, F), jnp.float32),
        pltpu.VMEM((SE2, F), jnp.float32),
        pltpu.VMEM_SHARED((NP, F), jnp.float32),
        pltpu.SemaphoreType.DMA((2,)),
        pltpu.SemaphoreType.DMA((2,)),
    ],
)
def _scatter_kernel(g_hbm, src_hbm, dst_hbm, zeros_hbm, out_hbm,
                    si_a, di_a, si_b, di_b, rows_a, rows_b,
                    acc_sh, sem_g, sem_s):
    c = lax.axis_index("c")
    s = lax.axis_index("s")
    r0 = s * RPT
    _init_acc(c, r0, g_hbm, zeros_hbm, acc_sh)
    e0 = (c * NS + s) * EPT
    plsc.subcore_barrier()

    def pair(q, carry):
        ea = e0 + q * (2 * SE2)
        eb = ea + SE2
        pltpu.sync_copy(src_hbm.at[pl.ds(ea, SE2)], si_a)
        pltpu.sync_copy(dst_hbm.at[pl.ds(ea, SE2)], di_a)
        ga = pltpu.async_copy(g_hbm.at[si_a], rows_a, sem_g.at[0])
        pltpu.sync_copy(src_hbm.at[pl.ds(eb, SE2)], si_b)
        pltpu.sync_copy(dst_hbm.at[pl.ds(eb, SE2)], di_b)
        gb = pltpu.async_copy(g_hbm.at[si_b], rows_b, sem_g.at[1])
        ga.wait()
        sa = pltpu.async_copy(rows_a, acc_sh.at[di_a], sem_s.at[0], add=True)
        gb.wait()
        sb = pltpu.async_copy(rows_b, acc_sh.at[di_b], sem_s.at[1], add=True)
        sa.wait()
        sb.wait()
        return carry

    lax.fori_loop(0, NPAIR, pair, 0)
    plsc.subcore_barrier()
    _write_out(c, r0, out_hbm, acc_sh)


def _tc1_body(x_ref, w_ref, degp_ref, g_ref, dinv_ref):
    d = degp_ref[0][:, 0:1] + degp_ref[1][:, 0:1]
    dv = lax.rsqrt(d)
    h = jnp.dot(x_ref[...], w_ref[...], preferred_element_type=jnp.float32)
    g_ref[...] = h * dv
    dinv_ref[...] = dv


def _tc_mid_body(acc_ref, dinv_ref, b_ref, w_ref, g_ref):
    dv = dinv_ref[...]
    h = (acc_ref[0] + acc_ref[1]) * dv + b_ref[...]
    h = jnp.maximum(h, 0.0)
    g_ref[...] = jnp.dot(h, w_ref[...], preferred_element_type=jnp.float32) * dv


def _tc_h3_body(acc_ref, dinv_ref, b_ref, h_ref, st_ref):
    i = pl.program_id(0)
    h = (acc_ref[0] + acc_ref[1]) * dinv_ref[...] + b_ref[...]
    h = jnp.maximum(h, 0.0)
    h_ref[...] = h
    s1 = jnp.sum(h, axis=0, keepdims=True)
    s2 = jnp.sum(h * h, axis=0, keepdims=True)
    st = jnp.concatenate([s1, s2], axis=0)

    @pl.when(i == 0)
    def _():
        st_ref[...] = st

    @pl.when(i != 0)
    def _():
        st_ref[...] += st


def _tc_head_body(h_ref, st_ref, gam_ref, bet_ref, wc_ref, bc_ref,
                  wr_ref, br_ref, o_ref):
    mean = st_ref[0:1] / float(N)
    var = st_ref[1:2] / float(N) - mean * mean
    xn = (h_ref[...] - mean) * lax.rsqrt(var + 1e-5) * gam_ref[...] + bet_ref[...]
    hc = jnp.dot(xn, wc_ref[...], preferred_element_type=jnp.float32) + bc_ref[...]
    hc = jnp.maximum(hc, 0.0)
    o_ref[...] = jnp.dot(hc, wr_ref[...], preferred_element_type=jnp.float32) + br_ref[...]


def _tc1(x, W1, degp):
    return pl.pallas_call(
        _tc1_body,
        grid=(NBLK,),
        in_specs=[
            pl.BlockSpec((CH, F), lambda i: (i, 0)),
            pl.BlockSpec((F, F), lambda i: (0, 0)),
            pl.BlockSpec((2, CH, F), lambda i: (0, i, 0)),
        ],
        out_specs=[
            pl.BlockSpec((CH, F), lambda i: (i, 0)),
            pl.BlockSpec((CH, 1), lambda i: (i, 0)),
        ],
        out_shape=[
            jax.ShapeDtypeStruct((NP, F), jnp.float32),
            jax.ShapeDtypeStruct((NP, 1), jnp.float32),
        ],
    )(x, W1, degp)


def _tc_mid(acc, dinv, b, W):
    return pl.pallas_call(
        _tc_mid_body,
        grid=(NBLK,),
        in_specs=[
            pl.BlockSpec((2, CH, F), lambda i: (0, i, 0)),
            pl.BlockSpec((CH, 1), lambda i: (i, 0)),
            pl.BlockSpec((1, F), lambda i: (0, 0)),
            pl.BlockSpec((F, F), lambda i: (0, 0)),
        ],
        out_specs=pl.BlockSpec((CH, F), lambda i: (i, 0)),
        out_shape=jax.ShapeDtypeStruct((NP, F), jnp.float32),
    )(acc, dinv, b, W)


def _tc_h3(acc, dinv, b):
    return pl.pallas_call(
        _tc_h3_body,
        grid=(NBLK,),
        in_specs=[
            pl.BlockSpec((2, CH, F), lambda i: (0, i, 0)),
            pl.BlockSpec((CH, 1), lambda i: (i, 0)),
            pl.BlockSpec((1, F), lambda i: (0, 0)),
        ],
        out_specs=[
            pl.BlockSpec((CH, F), lambda i: (i, 0)),
            pl.BlockSpec((2, F), lambda i: (0, 0)),
        ],
        out_shape=[
            jax.ShapeDtypeStruct((N, F), jnp.float32),
            jax.ShapeDtypeStruct((2, F), jnp.float32),
        ],
    )(acc, dinv, b)


def _tc_head(h3, stats, gamma, beta, Wc, bc, Wr, br):
    return pl.pallas_call(
        _tc_head_body,
        grid=(NBLK,),
        in_specs=[
            pl.BlockSpec((CH, F), lambda i: (i, 0)),
            pl.BlockSpec((2, F), lambda i: (0, 0)),
            pl.BlockSpec((1, F), lambda i: (0, 0)),
            pl.BlockSpec((1, F), lambda i: (0, 0)),
            pl.BlockSpec((F, F), lambda i: (0, 0)),
            pl.BlockSpec((1, F), lambda i: (0, 0)),
            pl.BlockSpec((F, O), lambda i: (0, 0)),
            pl.BlockSpec((1, O), lambda i: (0, 0)),
        ],
        out_specs=pl.BlockSpec((CH, O), lambda i: (i, 0)),
        out_shape=jax.ShapeDtypeStruct((N, O), jnp.float32),
    )(h3, stats, gamma, beta, Wc, bc, Wr, br)


def kernel(x, edge_index, W1, b1, W2, b2, W3, b3, gamma, beta, Wc, bc, Wr, br):
    pad_w = EPT - EDGES_PER_TILE
    src2 = jnp.pad(edge_index[0].reshape(NC * NS, EDGES_PER_TILE),
                   ((0, 0), (0, pad_w))).reshape(EROWS, SE)
    dst2 = jnp.pad(edge_index[1].reshape(NC * NS, EDGES_PER_TILE),
                   ((0, 0), (0, pad_w)), constant_values=SAC).reshape(EROWS, SE)
    zeros = jnp.zeros((NP, F), jnp.float32)
    ones_tab = jnp.ones((NP, F), jnp.float32)

    src1p = src2.reshape(EP)
    dst1p = dst2.reshape(EP)
    degp = _deg_kernel(dst2, ones_tab, zeros)
    g1, dinv = _tc1(x, W1, degp)
    acc1 = _scatter_kernel(g1, src1p, dst1p, zeros)
    g2 = _tc_mid(acc1, dinv, b1.reshape(1, F), W2)
    acc2 = _scatter_kernel(g2, src1p, dst1p, zeros)
    g3 = _tc_mid(acc2, dinv, b2.reshape(1, F), W3)
    acc3 = _scatter_kernel(g3, src1p, dst1p, zeros)
    h3, stats = _tc_h3(acc3, dinv, b3.reshape(1, F))
    return _tc_head(h3, stats, gamma.reshape(1, F), beta.reshape(1, F),
                    Wc, bc.reshape(1, F), Wr, br.reshape(1, O))


# speedup vs baseline: 1.0560x; 1.0560x over previous
import functools

import jax
import jax.numpy as jnp
from jax import lax
from jax.experimental import pallas as pl
from jax.experimental.pallas import tpu as pltpu
from jax.experimental.pallas import tpu_sc as plsc

N = 10000
NP = 10240
F = 128
E = 320000
O = 40

NC = 2
NS = 16
RPT = NP // NS
RC = 64
EDGES_PER_SC = E // NC
EDGES_PER_TILE = EDGES_PER_SC // NS
SE = 128
EP = 327680
SAC = 10000
EPT = EP // (NC * NS)
NCHUNK = EPT // SE
K = 8
NGROUP = NCHUNK // K
EROWS = EP // SE

CH = 1000
NBLK = N // CH

_MESH = plsc.VectorSubcoreMesh(core_axis_name="c", subcore_axis_name="s")


def _init_acc(c, r0, pos_hbm, zeros_hbm, acc_sh):
    @pl.when(c == 0)
    def _():
        pltpu.sync_copy(pos_hbm.at[pl.ds(r0, RPT)], acc_sh.at[pl.ds(r0, RPT)])

    @pl.when(c != 0)
    def _():
        pltpu.sync_copy(zeros_hbm.at[pl.ds(r0, RPT)], acc_sh.at[pl.ds(r0, RPT)])


def _write_out(c, r0, out_hbm, acc_sh):
    pltpu.sync_copy(acc_sh.at[pl.ds(r0, RPT)], out_hbm.at[c, pl.ds(r0, RPT)])


@functools.partial(
    pl.kernel,
    out_type=jax.ShapeDtypeStruct((NC, NP, F), jnp.float32),
    mesh=_MESH,
    scratch_types=[
        pltpu.VMEM((NCHUNK, SE), jnp.int32),
        pltpu.VMEM((SE, F), jnp.float32),
        pltpu.VMEM_SHARED((NP, F), jnp.float32),
        pltpu.SemaphoreType.DMA((4,)),
    ],
)
def _deg_kernel(dst2_hbm, ones_hbm, zeros_hbm, out_hbm,
                di_all, ones_v, acc_sh, sem_s):
    c = lax.axis_index("c")
    s = lax.axis_index("s")
    r0 = s * RPT
    _init_acc(c, r0, ones_hbm, zeros_hbm, acc_sh)
    pltpu.sync_copy(ones_hbm.at[pl.ds(0, SE)], ones_v)
    row0 = (c * NS + s) * NCHUNK
    pltpu.sync_copy(dst2_hbm.at[pl.ds(row0, NCHUNK)], di_all)
    plsc.subcore_barrier()

    def group(j0, carry):
        base = j0 * 4
        descs = [
            pltpu.async_copy(ones_v, acc_sh.at[di_all.at[base + b]],
                             sem_s.at[b], add=True)
            for b in range(4)
        ]
        for d in descs:
            d.wait()
        return carry

    lax.fori_loop(0, NCHUNK // 4, group, 0)
    plsc.subcore_barrier()
    _write_out(c, r0, out_hbm, acc_sh)


SE2 = 80
NCH2 = EPT // SE2
NPAIR = NCH2 // 2


@functools.partial(
    pl.kernel,
    out_type=jax.ShapeDtypeStruct((NC, NP, F), jnp.float32),
    mesh=_MESH,
    scratch_types=[
        pltpu.VMEM((SE2,), jnp.int32),
        pltpu.VMEM((SE2,), jnp.int32),
        pltpu.VMEM((SE2,), jnp.int32),
        pltpu.VMEM((SE2,), jnp.int32),
        pltpu.VMEM((SE2, F), jnp.float32),
        pltpu.VMEM((SE2, F), jnp.float32),
        pltpu.VMEM_SHARED((NP, F), jnp.float32),
        pltpu.SemaphoreType.DMA((2,)),
        pltpu.SemaphoreType.DMA((2,)),
    ],
)
def _scatter_kernel(g_hbm, src_hbm, dst_hbm, zeros_hbm, out_hbm,
                    si_a, di_a, si_b, di_b, rows_a, rows_b,
                    acc_sh, sem_g, sem_s):
    c = lax.axis_index("c")
    s = lax.axis_index("s")
    r0 = s * RPT
    _init_acc(c, r0, g_hbm, zeros_hbm, acc_sh)
    e0 = (c * NS + s) * EPT
    plsc.subcore_barrier()

    def pair(q, carry):
        ea = e0 + q * (2 * SE2)
        eb = ea + SE2
        pltpu.sync_copy(src_hbm.at[pl.ds(ea, SE2)], si_a)
        pltpu.sync_copy(dst_hbm.at[pl.ds(ea, SE2)], di_a)
        ga = pltpu.async_copy(g_hbm.at[si_a], rows_a, sem_g.at[0])
        pltpu.sync_copy(src_hbm.at[pl.ds(eb, SE2)], si_b)
        pltpu.sync_copy(dst_hbm.at[pl.ds(eb, SE2)], di_b)
        gb = pltpu.async_copy(g_hbm.at[si_b], rows_b, sem_g.at[1])
        ga.wait()
        sa = pltpu.async_copy(rows_a, acc_sh.at[di_a], sem_s.at[0], add=True)
        gb.wait()
        sb = pltpu.async_copy(rows_b, acc_sh.at[di_b], sem_s.at[1], add=True)
        sa.wait()
        sb.wait()
        return carry

    lax.fori_loop(0, NPAIR, pair, 0)
    plsc.subcore_barrier()
    _write_out(c, r0, out_hbm, acc_sh)


def _tc1_body(x_ref, w_ref, degp_ref, g_ref, dinv_ref):
    d = degp_ref[0][:, 0:1] + degp_ref[1][:, 0:1]
    dv = lax.rsqrt(d)
    h = jnp.dot(x_ref[...], w_ref[...], preferred_element_type=jnp.float32)
    g_ref[...] = h * dv
    dinv_ref[...] = dv


def _tc_mid_body(acc_ref, dinv_ref, b_ref, w_ref, g_ref):
    dv = dinv_ref[...]
    h = (acc_ref[0] + acc_ref[1]) * dv + b_ref[...]
    h = jnp.maximum(h, 0.0)
    g_ref[...] = jnp.dot(h, w_ref[...], preferred_element_type=jnp.float32) * dv


def _tc_h3_body(acc_ref, dinv_ref, b_ref, h_ref, st_ref):
    i = pl.program_id(0)
    h = (acc_ref[0] + acc_ref[1]) * dinv_ref[...] + b_ref[...]
    h = jnp.maximum(h, 0.0)
    h_ref[...] = h
    s1 = jnp.sum(h, axis=0, keepdims=True)
    s2 = jnp.sum(h * h, axis=0, keepdims=True)
    st = jnp.concatenate([s1, s2], axis=0)

    @pl.when(i == 0)
    def _():
        st_ref[...] = st

    @pl.when(i != 0)
    def _():
        st_ref[...] += st


def _tc_head_body(h_ref, st_ref, gam_ref, bet_ref, wc_ref, bc_ref,
                  wr_ref, br_ref, o_ref):
    mean = st_ref[0:1] / float(N)
    var = st_ref[1:2] / float(N) - mean * mean
    xn = (h_ref[...] - mean) * lax.rsqrt(var + 1e-5) * gam_ref[...] + bet_ref[...]
    hc = jnp.dot(xn, wc_ref[...], preferred_element_type=jnp.float32) + bc_ref[...]
    hc = jnp.maximum(hc, 0.0)
    o_ref[...] = jnp.dot(hc, wr_ref[...], preferred_element_type=jnp.float32) + br_ref[...]


def _tc1(x, W1, degp):
    return pl.pallas_call(
        _tc1_body,
        grid=(NBLK,),
        in_specs=[
            pl.BlockSpec((CH, F), lambda i: (i, 0)),
            pl.BlockSpec((F, F), lambda i: (0, 0)),
            pl.BlockSpec((2, CH, F), lambda i: (0, i, 0)),
        ],
        out_specs=[
            pl.BlockSpec((CH, F), lambda i: (i, 0)),
            pl.BlockSpec((CH, 1), lambda i: (i, 0)),
        ],
        out_shape=[
            jax.ShapeDtypeStruct((NP, F), jnp.float32),
            jax.ShapeDtypeStruct((NP, 1), jnp.float32),
        ],
    )(x, W1, degp)


def _tc_mid(acc, dinv, b, W):
    return pl.pallas_call(
        _tc_mid_body,
        grid=(NBLK,),
        in_specs=[
            pl.BlockSpec((2, CH, F), lambda i: (0, i, 0)),
            pl.BlockSpec((CH, 1), lambda i: (i, 0)),
            pl.BlockSpec((1, F), lambda i: (0, 0)),
            pl.BlockSpec((F, F), lambda i: (0, 0)),
        ],
        out_specs=pl.BlockSpec((CH, F), lambda i: (i, 0)),
        out_shape=jax.ShapeDtypeStruct((NP, F), jnp.float32),
    )(acc, dinv, b, W)


def _tc_h3(acc, dinv, b):
    return pl.pallas_call(
        _tc_h3_body,
        grid=(NBLK,),
        in_specs=[
            pl.BlockSpec((2, CH, F), lambda i: (0, i, 0)),
            pl.BlockSpec((CH, 1), lambda i: (i, 0)),
            pl.BlockSpec((1, F), lambda i: (0, 0)),
        ],
        out_specs=[
            pl.BlockSpec((CH, F), lambda i: (i, 0)),
            pl.BlockSpec((2, F), lambda i: (0, 0)),
        ],
        out_shape=[
            jax.ShapeDtypeStruct((N, F), jnp.float32),
            jax.ShapeDtypeStruct((2, F), jnp.float32),
        ],
    )(acc, dinv, b)


def _tc_head(h3, stats, gamma, beta, Wc, bc, Wr, br):
    return pl.pallas_call(
        _tc_head_body,
        grid=(NBLK,),
        in_specs=[
            pl.BlockSpec((CH, F), lambda i: (i, 0)),
            pl.BlockSpec((2, F), lambda i: (0, 0)),
            pl.BlockSpec((1, F), lambda i: (0, 0)),
            pl.BlockSpec((1, F), lambda i: (0, 0)),
            pl.BlockSpec((F, F), lambda i: (0, 0)),
            pl.BlockSpec((1, F), lambda i: (0, 0)),
            pl.BlockSpec((F, O), lambda i: (0, 0)),
            pl.BlockSpec((1, O), lambda i: (0, 0)),
        ],
        out_specs=pl.BlockSpec((CH, O), lambda i: (i, 0)),
        out_shape=jax.ShapeDtypeStruct((N, O), jnp.float32),
    )(h3, stats, gamma, beta, Wc, bc, Wr, br)


def kernel(x, edge_index, W1, b1, W2, b2, W3, b3, gamma, beta, Wc, bc, Wr, br):
    pad_w = EPT - EDGES_PER_TILE
    src2 = jnp.pad(edge_index[0].reshape(NC * NS, EDGES_PER_TILE),
                   ((0, 0), (0, pad_w))).reshape(EROWS, SE)
    dst2 = jnp.pad(edge_index[1].reshape(NC * NS, EDGES_PER_TILE),
                   ((0, 0), (0, pad_w)), constant_values=SAC).reshape(EROWS, SE)
    zeros = jnp.zeros((NP, F), jnp.float32)
    ones_tab = jnp.ones((NP, F), jnp.float32)

    src1p = src2.reshape(EP)
    dst1p = dst2.reshape(EP)
    degp = _deg_kernel(dst2, ones_tab, zeros)
    g1, dinv = _tc1(x, W1, degp)
    acc1 = _scatter_kernel(g1, src1p, dst1p, zeros)
    g2 = _tc_mid(acc1, dinv, b1.reshape(1, F), W2)
    acc2 = _scatter_kernel(g2, src1p, dst1p, zeros)
    g3 = _tc_mid(acc2, dinv, b2.reshape(1, F), W3)
    acc3 = _scatter_kernel(g3, src1p, dst1p, zeros)
    h3, stats = _tc_h3(acc3, dinv, b3.reshape(1, F))
    return _tc_head(h3, stats, gamma.reshape(1, F), beta.reshape(1, F),
                    Wc, bc.reshape(1, F), Wr, br.reshape(1, O))


# speedup vs baseline: 2.0877x; 1.9769x over previous
import functools

import jax
import jax.numpy as jnp
from jax import lax
from jax.experimental import pallas as pl
from jax.experimental.pallas import tpu as pltpu
from jax.experimental.pallas import tpu_sc as plsc

N = 10000
NP = 10240
F = 128
E = 320000
O = 40

NC = 2
NS = 16
RPT = NP // NS
RC = 64
EDGES_PER_SC = E // NC
EDGES_PER_TILE = EDGES_PER_SC // NS
SE = 128
EP = 327680
SAC = 10000
EPT = EP // (NC * NS)
NCHUNK = EPT // SE
K = 8
NGROUP = NCHUNK // K
EROWS = EP // SE

CH = 1000
NBLK = N // CH

_MESH = plsc.VectorSubcoreMesh(core_axis_name="c", subcore_axis_name="s")


def _init_acc(c, r0, pos_hbm, zeros_hbm, acc_sh):
    @pl.when(c == 0)
    def _():
        pltpu.sync_copy(pos_hbm.at[pl.ds(r0, RPT)], acc_sh.at[pl.ds(r0, RPT)])

    @pl.when(c != 0)
    def _():
        pltpu.sync_copy(zeros_hbm.at[pl.ds(r0, RPT)], acc_sh.at[pl.ds(r0, RPT)])


def _write_out(c, r0, out_hbm, acc_sh):
    pltpu.sync_copy(acc_sh.at[pl.ds(r0, RPT)], out_hbm.at[c, pl.ds(r0, RPT)])


@functools.partial(
    pl.kernel,
    out_type=jax.ShapeDtypeStruct((NC, NP, F), jnp.float32),
    mesh=_MESH,
    scratch_types=[
        pltpu.VMEM((NCHUNK, SE), jnp.int32),
        pltpu.VMEM((SE, F), jnp.float32),
        pltpu.VMEM_SHARED((NP, F), jnp.float32),
        pltpu.SemaphoreType.DMA((4,)),
    ],
)
def _deg_kernel(dst2_hbm, ones_hbm, zeros_hbm, out_hbm,
                di_all, ones_v, acc_sh, sem_s):
    c = lax.axis_index("c")
    s = lax.axis_index("s")
    r0 = s * RPT
    _init_acc(c, r0, ones_hbm, zeros_hbm, acc_sh)
    pltpu.sync_copy(ones_hbm.at[pl.ds(0, SE)], ones_v)
    row0 = (c * NS + s) * NCHUNK
    pltpu.sync_copy(dst2_hbm.at[pl.ds(row0, NCHUNK)], di_all)
    plsc.subcore_barrier()

    def group(j0, carry):
        base = j0 * 4
        descs = [
            pltpu.async_copy(ones_v, acc_sh.at[di_all.at[base + b]],
                             sem_s.at[b], add=True)
            for b in range(4)
        ]
        for d in descs:
            d.wait()
        return carry

    lax.fori_loop(0, NCHUNK // 4, group, 0)
    plsc.subcore_barrier()
    _write_out(c, r0, out_hbm, acc_sh)


SE2 = 80
NCH2 = EPT // SE2
NPAIR = NCH2 // 2


@functools.partial(
    pl.kernel,
    out_type=jax.ShapeDtypeStruct((NC, NP, F), jnp.float32),
    mesh=_MESH,
    scratch_types=[
        pltpu.VMEM((SE2,), jnp.int32),
        pltpu.VMEM((SE2,), jnp.int32),
        pltpu.VMEM((SE2,), jnp.int32),
        pltpu.VMEM((SE2,), jnp.int32),
        pltpu.VMEM((SE2, F), jnp.float32),
        pltpu.VMEM((SE2, F), jnp.float32),
        pltpu.VMEM_SHARED((NP, F), jnp.float32),
        pltpu.SemaphoreType.DMA((2,)),
        pltpu.SemaphoreType.DMA((2,)),
    ],
)
def _scatter_kernel(g_hbm, src_hbm, dst_hbm, zeros_hbm, out_hbm,
                    si_a, di_a, si_b, di_b, rows_a, rows_b,
                    acc_sh, sem_g, sem_s):
    c = lax.axis_index("c")
    s = lax.axis_index("s")
    r0 = s * RPT
    _init_acc(c, r0, g_hbm, zeros_hbm, acc_sh)
    e0 = (c * NS + s) * EPT
    plsc.subcore_barrier()

    def pair(q, carry):
        ea = e0 + q * (2 * SE2)
        eb = ea + SE2
        pltpu.sync_copy(src_hbm.at[pl.ds(ea, SE2)], si_a)
        pltpu.sync_copy(dst_hbm.at[pl.ds(ea, SE2)], di_a)
        ga = pltpu.async_copy(g_hbm.at[si_a], rows_a, sem_g.at[0])
        pltpu.sync_copy(src_hbm.at[pl.ds(eb, SE2)], si_b)
        pltpu.sync_copy(dst_hbm.at[pl.ds(eb, SE2)], di_b)
        gb = pltpu.async_copy(g_hbm.at[si_b], rows_b, sem_g.at[1])
        ga.wait()
        sa = pltpu.async_copy(rows_a, acc_sh.at[di_a], sem_s.at[0], add=True)
        gb.wait()
        sb = pltpu.async_copy(rows_b, acc_sh.at[di_b], sem_s.at[1], add=True)
        sa.wait()
        sb.wait()
        return carry

    lax.fori_loop(0, NPAIR, pair, 0)
    plsc.subcore_barrier()
    _write_out(c, r0, out_hbm, acc_sh)


def _tc1_body(x_ref, w_ref, degp_ref, g_ref, dinv_ref):
    d = degp_ref[0][:, 0:1] + degp_ref[1][:, 0:1]
    dv = lax.rsqrt(d)
    h = jnp.dot(x_ref[...], w_ref[...], preferred_element_type=jnp.float32)
    g_ref[...] = h * dv
    dinv_ref[...] = dv


def _tc_mid_body(acc_ref, dinv_ref, b_ref, w_ref, g_ref):
    dv = dinv_ref[...]
    h = (acc_ref[0] + acc_ref[1]) * dv + b_ref[...]
    h = jnp.maximum(h, 0.0)
    g_ref[...] = jnp.dot(h, w_ref[...], preferred_element_type=jnp.float32) * dv


def _tc_h3_body(acc_ref, dinv_ref, b_ref, h_ref, st_ref):
    i = pl.program_id(0)
    h = (acc_ref[0] + acc_ref[1]) * dinv_ref[...] + b_ref[...]
    h = jnp.maximum(h, 0.0)
    h_ref[...] = h
    s1 = jnp.sum(h, axis=0, keepdims=True)
    s2 = jnp.sum(h * h, axis=0, keepdims=True)
    st = jnp.concatenate([s1, s2], axis=0)

    @pl.when(i == 0)
    def _():
        st_ref[...] = st

    @pl.when(i != 0)
    def _():
        st_ref[...] += st


def _tc_head_body(h_ref, st_ref, gam_ref, bet_ref, wc_ref, bc_ref,
                  wr_ref, br_ref, o_ref):
    mean = st_ref[0:1] / float(N)
    var = st_ref[1:2] / float(N) - mean * mean
    xn = (h_ref[...] - mean) * lax.rsqrt(var + 1e-5) * gam_ref[...] + bet_ref[...]
    hc = jnp.dot(xn, wc_ref[...], preferred_element_type=jnp.float32) + bc_ref[...]
    hc = jnp.maximum(hc, 0.0)
    o_ref[...] = jnp.dot(hc, wr_ref[...], preferred_element_type=jnp.float32) + br_ref[...]


def _tc1(x, W1, degp):
    return pl.pallas_call(
        _tc1_body,
        grid=(NBLK,),
        in_specs=[
            pl.BlockSpec((CH, F), lambda i: (i, 0)),
            pl.BlockSpec((F, F), lambda i: (0, 0)),
            pl.BlockSpec((2, CH, F), lambda i: (0, i, 0)),
        ],
        out_specs=[
            pl.BlockSpec((CH, F), lambda i: (i, 0)),
            pl.BlockSpec((CH, 1), lambda i: (i, 0)),
        ],
        out_shape=[
            jax.ShapeDtypeStruct((NP, F), jnp.float32),
            jax.ShapeDtypeStruct((NP, 1), jnp.float32),
        ],
    )(x, W1, degp)


def _tc_mid(acc, dinv, b, W):
    return pl.pallas_call(
        _tc_mid_body,
        grid=(NBLK,),
        in_specs=[
            pl.BlockSpec((2, CH, F), lambda i: (0, i, 0)),
            pl.BlockSpec((CH, 1), lambda i: (i, 0)),
            pl.BlockSpec((1, F), lambda i: (0, 0)),
            pl.BlockSpec((F, F), lambda i: (0, 0)),
        ],
        out_specs=pl.BlockSpec((CH, F), lambda i: (i, 0)),
        out_shape=jax.ShapeDtypeStruct((NP, F), jnp.float32),
    )(acc, dinv, b, W)


def _tc_h3(acc, dinv, b):
    return pl.pallas_call(
        _tc_h3_body,
        grid=(NBLK,),
        in_specs=[
            pl.BlockSpec((2, CH, F), lambda i: (0, i, 0)),
            pl.BlockSpec((CH, 1), lambda i: (i, 0)),
            pl.BlockSpec((1, F), lambda i: (0, 0)),
        ],
        out_specs=[
            pl.BlockSpec((CH, F), lambda i: (i, 0)),
            pl.BlockSpec((2, F), lambda i: (0, 0)),
        ],
        out_shape=[
            jax.ShapeDtypeStruct((N, F), jnp.float32),
            jax.ShapeDtypeStruct((2, F), jnp.float32),
        ],
    )(acc, dinv, b)


def _tc_head(h3, stats, gamma, beta, Wc, bc, Wr, br):
    return pl.pallas_call(
        _tc_head_body,
        grid=(NBLK,),
        in_specs=[
            pl.BlockSpec((CH, F), lambda i: (i, 0)),
            pl.BlockSpec((2, F), lambda i: (0, 0)),
            pl.BlockSpec((1, F), lambda i: (0, 0)),
            pl.BlockSpec((1, F), lambda i: (0, 0)),
            pl.BlockSpec((F, F), lambda i: (0, 0)),
            pl.BlockSpec((1, F), lambda i: (0, 0)),
            pl.BlockSpec((F, O), lambda i: (0, 0)),
            pl.BlockSpec((1, O), lambda i: (0, 0)),
        ],
        out_specs=pl.BlockSpec((CH, O), lambda i: (i, 0)),
        out_shape=jax.ShapeDtypeStruct((N, O), jnp.float32),
    )(h3, stats, gamma, beta, Wc, bc, Wr, br)


def kernel(x, edge_index, W1, b1, W2, b2, W3, b3, gamma, beta, Wc, bc, Wr, br):
    pad_w = EPT - EDGES_PER_TILE
    pad_src = jnp.broadcast_to(jnp.arange(pad_w, dtype=jnp.int32),
                               (NC * NS, pad_w))
    pad_dst = pad_src + SAC
    src2 = jnp.concatenate(
        [edge_index[0].reshape(NC * NS, EDGES_PER_TILE), pad_src],
        axis=1).reshape(EROWS, SE)
    dst2 = jnp.concatenate(
        [edge_index[1].reshape(NC * NS, EDGES_PER_TILE), pad_dst],
        axis=1).reshape(EROWS, SE)
    zeros = jnp.zeros((NP, F), jnp.float32)
    ones_tab = jnp.ones((NP, F), jnp.float32)

    src1p = src2.reshape(EP)
    dst1p = dst2.reshape(EP)
    degp = _deg_kernel(dst2, ones_tab, zeros)
    g1, dinv = _tc1(x, W1, degp)
    acc1 = _scatter_kernel(g1, src1p, dst1p, zeros)
    g2 = _tc_mid(acc1, dinv, b1.reshape(1, F), W2)
    acc2 = _scatter_kernel(g2, src1p, dst1p, zeros)
    g3 = _tc_mid(acc2, dinv, b2.reshape(1, F), W3)
    acc3 = _scatter_kernel(g3, src1p, dst1p, zeros)
    h3, stats = _tc_h3(acc3, dinv, b3.reshape(1, F))
    return _tc_head(h3, stats, gamma.reshape(1, F), beta.reshape(1, F),
                    Wc, bc.reshape(1, F), Wr, br.reshape(1, O))


# speedup vs baseline: 2.5278x; 1.2108x over previous
import functools

import jax
import jax.numpy as jnp
from jax import lax
from jax.experimental import pallas as pl
from jax.experimental.pallas import tpu as pltpu
from jax.experimental.pallas import tpu_sc as plsc

N = 10000
NP = 10240
F = 128
E = 320000
O = 40

NC = 2
NS = 16
RPT = NP // NS
RC = 64
EDGES_PER_SC = E // NC
EDGES_PER_TILE = EDGES_PER_SC // NS
SE = 128
EP = 327680
SAC = 10000
EPT = EP // (NC * NS)
NCHUNK = EPT // SE
K = 8
NGROUP = NCHUNK // K
EROWS = EP // SE

CH = 1000
NBLK = N // CH

_MESH = plsc.VectorSubcoreMesh(core_axis_name="c", subcore_axis_name="s")


def _init_acc(c, r0, pos_hbm, zeros_hbm, acc_sh):
    @pl.when(c == 0)
    def _():
        pltpu.sync_copy(pos_hbm.at[pl.ds(r0, RPT)], acc_sh.at[pl.ds(r0, RPT)])

    @pl.when(c != 0)
    def _():
        pltpu.sync_copy(zeros_hbm.at[pl.ds(r0, RPT)], acc_sh.at[pl.ds(r0, RPT)])


def _write_out(c, r0, out_hbm, acc_sh):
    pltpu.sync_copy(acc_sh.at[pl.ds(r0, RPT)], out_hbm.at[c, pl.ds(r0, RPT)])


@functools.partial(
    pl.kernel,
    out_type=jax.ShapeDtypeStruct((NC, NP, F), jnp.float32),
    mesh=_MESH,
    scratch_types=[
        pltpu.VMEM((NCHUNK, SE), jnp.int32),
        pltpu.VMEM((SE, F), jnp.float32),
        pltpu.VMEM_SHARED((NP, F), jnp.float32),
        pltpu.SemaphoreType.DMA((4,)),
    ],
)
def _deg_kernel(dst2_hbm, ones_hbm, zeros_hbm, out_hbm,
                di_all, ones_v, acc_sh, sem_s):
    c = lax.axis_index("c")
    s = lax.axis_index("s")
    r0 = s * RPT
    _init_acc(c, r0, ones_hbm, zeros_hbm, acc_sh)
    pltpu.sync_copy(ones_hbm.at[pl.ds(0, SE)], ones_v)
    row0 = (c * NS + s) * NCHUNK
    pltpu.sync_copy(dst2_hbm.at[pl.ds(row0, NCHUNK)], di_all)
    plsc.subcore_barrier()

    def group(j0, carry):
        base = j0 * 4
        descs = [
            pltpu.async_copy(ones_v, acc_sh.at[di_all.at[base + b]],
                             sem_s.at[b], add=True)
            for b in range(4)
        ]
        for d in descs:
            d.wait()
        return carry

    lax.fori_loop(0, NCHUNK // 4, group, 0)
    plsc.subcore_barrier()
    _write_out(c, r0, out_hbm, acc_sh)


SE2 = 128
NCH2 = EPT // SE2
NPAIR = NCH2 // 2


@functools.partial(
    pl.kernel,
    out_type=jax.ShapeDtypeStruct((NC, NP, F), jnp.float32),
    mesh=_MESH,
    scratch_types=[
        pltpu.VMEM((SE2,), jnp.int32),
        pltpu.VMEM((SE2,), jnp.int32),
        pltpu.VMEM((SE2,), jnp.int32),
        pltpu.VMEM((SE2,), jnp.int32),
        pltpu.VMEM((SE2, F), jnp.float32),
        pltpu.VMEM((SE2, F), jnp.float32),
        pltpu.VMEM_SHARED((NP, F), jnp.float32),
        pltpu.SemaphoreType.DMA((2,)),
        pltpu.SemaphoreType.DMA((2,)),
    ],
)
def _scatter_kernel(g_hbm, src_hbm, dst_hbm, zeros_hbm, out_hbm,
                    si_a, di_a, si_b, di_b, rows_a, rows_b,
                    acc_sh, sem_g, sem_s):
    c = lax.axis_index("c")
    s = lax.axis_index("s")
    r0 = s * RPT
    _init_acc(c, r0, g_hbm, zeros_hbm, acc_sh)
    e0 = (c * NS + s) * EPT
    plsc.subcore_barrier()

    def pair(q, carry):
        ea = e0 + q * (2 * SE2)
        eb = ea + SE2
        pltpu.sync_copy(src_hbm.at[pl.ds(ea, SE2)], si_a)
        pltpu.sync_copy(dst_hbm.at[pl.ds(ea, SE2)], di_a)
        ga = pltpu.async_copy(g_hbm.at[si_a], rows_a, sem_g.at[0])
        pltpu.sync_copy(src_hbm.at[pl.ds(eb, SE2)], si_b)
        pltpu.sync_copy(dst_hbm.at[pl.ds(eb, SE2)], di_b)
        gb = pltpu.async_copy(g_hbm.at[si_b], rows_b, sem_g.at[1])
        ga.wait()
        sa = pltpu.async_copy(rows_a, acc_sh.at[di_a], sem_s.at[0], add=True)
        gb.wait()
        sb = pltpu.async_copy(rows_b, acc_sh.at[di_b], sem_s.at[1], add=True)
        sa.wait()
        sb.wait()
        return carry

    lax.fori_loop(0, NPAIR, pair, 0)
    plsc.subcore_barrier()
    _write_out(c, r0, out_hbm, acc_sh)


def _tc1_body(x_ref, w_ref, degp_ref, g_ref, dinv_ref):
    d = degp_ref[0][:, 0:1] + degp_ref[1][:, 0:1]
    dv = lax.rsqrt(d)
    h = jnp.dot(x_ref[...], w_ref[...], preferred_element_type=jnp.float32)
    g_ref[...] = h * dv
    dinv_ref[...] = dv


def _tc_mid_body(acc_ref, dinv_ref, b_ref, w_ref, g_ref):
    dv = dinv_ref[...]
    h = (acc_ref[0] + acc_ref[1]) * dv + b_ref[...]
    h = jnp.maximum(h, 0.0)
    g_ref[...] = jnp.dot(h, w_ref[...], preferred_element_type=jnp.float32) * dv


def _tc_h3_body(acc_ref, dinv_ref, b_ref, h_ref, st_ref):
    i = pl.program_id(0)
    h = (acc_ref[0] + acc_ref[1]) * dinv_ref[...] + b_ref[...]
    h = jnp.maximum(h, 0.0)
    h_ref[...] = h
    s1 = jnp.sum(h, axis=0, keepdims=True)
    s2 = jnp.sum(h * h, axis=0, keepdims=True)
    st = jnp.concatenate([s1, s2], axis=0)

    @pl.when(i == 0)
    def _():
        st_ref[...] = st

    @pl.when(i != 0)
    def _():
        st_ref[...] += st


def _tc_head_body(h_ref, st_ref, gam_ref, bet_ref, wc_ref, bc_ref,
                  wr_ref, br_ref, o_ref):
    mean = st_ref[0:1] / float(N)
    var = st_ref[1:2] / float(N) - mean * mean
    xn = (h_ref[...] - mean) * lax.rsqrt(var + 1e-5) * gam_ref[...] + bet_ref[...]
    hc = jnp.dot(xn, wc_ref[...], preferred_element_type=jnp.float32) + bc_ref[...]
    hc = jnp.maximum(hc, 0.0)
    o_ref[...] = jnp.dot(hc, wr_ref[...], preferred_element_type=jnp.float32) + br_ref[...]


def _tc1(x, W1, degp):
    return pl.pallas_call(
        _tc1_body,
        grid=(NBLK,),
        in_specs=[
            pl.BlockSpec((CH, F), lambda i: (i, 0)),
            pl.BlockSpec((F, F), lambda i: (0, 0)),
            pl.BlockSpec((2, CH, F), lambda i: (0, i, 0)),
        ],
        out_specs=[
            pl.BlockSpec((CH, F), lambda i: (i, 0)),
            pl.BlockSpec((CH, 1), lambda i: (i, 0)),
        ],
        out_shape=[
            jax.ShapeDtypeStruct((NP, F), jnp.float32),
            jax.ShapeDtypeStruct((NP, 1), jnp.float32),
        ],
    )(x, W1, degp)


def _tc_mid(acc, dinv, b, W):
    return pl.pallas_call(
        _tc_mid_body,
        grid=(NBLK,),
        in_specs=[
            pl.BlockSpec((2, CH, F), lambda i: (0, i, 0)),
            pl.BlockSpec((CH, 1), lambda i: (i, 0)),
            pl.BlockSpec((1, F), lambda i: (0, 0)),
            pl.BlockSpec((F, F), lambda i: (0, 0)),
        ],
        out_specs=pl.BlockSpec((CH, F), lambda i: (i, 0)),
        out_shape=jax.ShapeDtypeStruct((NP, F), jnp.float32),
    )(acc, dinv, b, W)


def _tc_h3(acc, dinv, b):
    return pl.pallas_call(
        _tc_h3_body,
        grid=(NBLK,),
        in_specs=[
            pl.BlockSpec((2, CH, F), lambda i: (0, i, 0)),
            pl.BlockSpec((CH, 1), lambda i: (i, 0)),
            pl.BlockSpec((1, F), lambda i: (0, 0)),
        ],
        out_specs=[
            pl.BlockSpec((CH, F), lambda i: (i, 0)),
            pl.BlockSpec((2, F), lambda i: (0, 0)),
        ],
        out_shape=[
            jax.ShapeDtypeStruct((N, F), jnp.float32),
            jax.ShapeDtypeStruct((2, F), jnp.float32),
        ],
    )(acc, dinv, b)


def _tc_head(h3, stats, gamma, beta, Wc, bc, Wr, br):
    return pl.pallas_call(
        _tc_head_body,
        grid=(NBLK,),
        in_specs=[
            pl.BlockSpec((CH, F), lambda i: (i, 0)),
            pl.BlockSpec((2, F), lambda i: (0, 0)),
            pl.BlockSpec((1, F), lambda i: (0, 0)),
            pl.BlockSpec((1, F), lambda i: (0, 0)),
            pl.BlockSpec((F, F), lambda i: (0, 0)),
            pl.BlockSpec((1, F), lambda i: (0, 0)),
            pl.BlockSpec((F, O), lambda i: (0, 0)),
            pl.BlockSpec((1, O), lambda i: (0, 0)),
        ],
        out_specs=pl.BlockSpec((CH, O), lambda i: (i, 0)),
        out_shape=jax.ShapeDtypeStruct((N, O), jnp.float32),
    )(h3, stats, gamma, beta, Wc, bc, Wr, br)


def kernel(x, edge_index, W1, b1, W2, b2, W3, b3, gamma, beta, Wc, bc, Wr, br):
    pad_w = EPT - EDGES_PER_TILE
    pad_src = jnp.broadcast_to(jnp.arange(pad_w, dtype=jnp.int32),
                               (NC * NS, pad_w))
    pad_dst = pad_src + SAC
    src2 = jnp.concatenate(
        [edge_index[0].reshape(NC * NS, EDGES_PER_TILE), pad_src],
        axis=1).reshape(EROWS, SE)
    dst2 = jnp.concatenate(
        [edge_index[1].reshape(NC * NS, EDGES_PER_TILE), pad_dst],
        axis=1).reshape(EROWS, SE)
    zeros = jnp.zeros((NP, F), jnp.float32)
    ones_tab = jnp.ones((NP, F), jnp.float32)

    src1p = src2.reshape(EP)
    dst1p = dst2.reshape(EP)
    degp = _deg_kernel(dst2, ones_tab, zeros)
    g1, dinv = _tc1(x, W1, degp)
    acc1 = _scatter_kernel(g1, src1p, dst1p, zeros)
    g2 = _tc_mid(acc1, dinv, b1.reshape(1, F), W2)
    acc2 = _scatter_kernel(g2, src1p, dst1p, zeros)
    g3 = _tc_mid(acc2, dinv, b2.reshape(1, F), W3)
    acc3 = _scatter_kernel(g3, src1p, dst1p, zeros)
    h3, stats = _tc_h3(acc3, dinv, b3.reshape(1, F))
    return _tc_head(h3, stats, gamma.reshape(1, F), beta.reshape(1, F),
                    Wc, bc.reshape(1, F), Wr, br.reshape(1, O))


# speedup vs baseline: 2.6340x; 1.0420x over previous
import functools

import jax
import jax.numpy as jnp
from jax import lax
from jax.experimental import pallas as pl
from jax.experimental.pallas import tpu as pltpu
from jax.experimental.pallas import tpu_sc as plsc

N = 10000
NP = 10240
F = 128
E = 320000
O = 40

NC = 2
NS = 16
RPT = NP // NS
RC = 64
EDGES_PER_SC = E // NC
EDGES_PER_TILE = EDGES_PER_SC // NS
SE = 128
EP = 327680
SAC = 10000
EPT = EP // (NC * NS)
NCHUNK = EPT // SE
K = 8
NGROUP = NCHUNK // K
EROWS = EP // SE

CH = 1000
NBLK = N // CH

_MESH = plsc.VectorSubcoreMesh(core_axis_name="c", subcore_axis_name="s")


def _init_acc(c, r0, pos_hbm, zeros_hbm, acc_sh):
    @pl.when(c == 0)
    def _():
        pltpu.sync_copy(pos_hbm.at[pl.ds(r0, RPT)], acc_sh.at[pl.ds(r0, RPT)])

    @pl.when(c != 0)
    def _():
        pltpu.sync_copy(zeros_hbm.at[pl.ds(r0, RPT)], acc_sh.at[pl.ds(r0, RPT)])


def _write_out(c, r0, out_hbm, acc_sh):
    pltpu.sync_copy(acc_sh.at[pl.ds(r0, RPT)], out_hbm.at[c, pl.ds(r0, RPT)])


@functools.partial(
    pl.kernel,
    out_type=jax.ShapeDtypeStruct((NC, NP, F), jnp.float32),
    mesh=_MESH,
    scratch_types=[
        pltpu.VMEM((NCHUNK, SE), jnp.int32),
        pltpu.VMEM((SE, F), jnp.float32),
        pltpu.VMEM_SHARED((NP, F), jnp.float32),
        pltpu.SemaphoreType.DMA((4,)),
    ],
)
def _deg_kernel(dst2_hbm, ones_hbm, zeros_hbm, out_hbm,
                di_all, ones_v, acc_sh, sem_s):
    c = lax.axis_index("c")
    s = lax.axis_index("s")
    r0 = s * RPT
    _init_acc(c, r0, ones_hbm, zeros_hbm, acc_sh)
    pltpu.sync_copy(ones_hbm.at[pl.ds(0, SE)], ones_v)
    row0 = (c * NS + s) * NCHUNK
    pltpu.sync_copy(dst2_hbm.at[pl.ds(row0, NCHUNK)], di_all)
    plsc.subcore_barrier()

    def group(j0, carry):
        base = j0 * 4
        descs = [
            pltpu.async_copy(ones_v, acc_sh.at[di_all.at[base + b]],
                             sem_s.at[b], add=True)
            for b in range(4)
        ]
        for d in descs:
            d.wait()
        return carry

    lax.fori_loop(0, NCHUNK // 4, group, 0)
    plsc.subcore_barrier()
    _write_out(c, r0, out_hbm, acc_sh)


SE2 = 128
NCH2 = EPT // SE2
NPAIR = NCH2 // 2


@functools.partial(
    pl.kernel,
    out_type=jax.ShapeDtypeStruct((NC, NP, F), jnp.float32),
    mesh=_MESH,
    scratch_types=[
        pltpu.VMEM((SE2,), jnp.int32),
        pltpu.VMEM((SE2,), jnp.int32),
        pltpu.VMEM((SE2,), jnp.int32),
        pltpu.VMEM((SE2,), jnp.int32),
        pltpu.VMEM((SE2, F), jnp.float32),
        pltpu.VMEM((SE2, F), jnp.float32),
        pltpu.VMEM_SHARED((NP, F), jnp.float32),
        pltpu.SemaphoreType.DMA((2,)),
        pltpu.SemaphoreType.DMA((2,)),
        pltpu.SemaphoreType.DMA((4,)),
    ],
)
def _scatter_kernel(g_hbm, src_hbm, dst_hbm, zeros_hbm, out_hbm,
                    si_a, di_a, si_b, di_b, rows_a, rows_b,
                    acc_sh, sem_g, sem_s, sem_i):
    c = lax.axis_index("c")
    s = lax.axis_index("s")
    r0 = s * RPT
    _init_acc(c, r0, g_hbm, zeros_hbm, acc_sh)
    e0 = (c * NS + s) * EPT
    plsc.subcore_barrier()

    def pair(q, carry):
        ea = e0 + q * (2 * SE2)
        eb = ea + SE2
        i1 = pltpu.async_copy(src_hbm.at[pl.ds(ea, SE2)], si_a, sem_i.at[0])
        i2 = pltpu.async_copy(dst_hbm.at[pl.ds(ea, SE2)], di_a, sem_i.at[1])
        i3 = pltpu.async_copy(src_hbm.at[pl.ds(eb, SE2)], si_b, sem_i.at[2])
        i4 = pltpu.async_copy(dst_hbm.at[pl.ds(eb, SE2)], di_b, sem_i.at[3])
        i1.wait()
        ga = pltpu.async_copy(g_hbm.at[si_a], rows_a, sem_g.at[0])
        i3.wait()
        gb = pltpu.async_copy(g_hbm.at[si_b], rows_b, sem_g.at[1])
        ga.wait()
        i2.wait()
        sa = pltpu.async_copy(rows_a, acc_sh.at[di_a], sem_s.at[0], add=True)
        gb.wait()
        i4.wait()
        sb = pltpu.async_copy(rows_b, acc_sh.at[di_b], sem_s.at[1], add=True)
        sa.wait()
        sb.wait()
        return carry

    lax.fori_loop(0, NPAIR, pair, 0)
    plsc.subcore_barrier()
    _write_out(c, r0, out_hbm, acc_sh)


def _tc_mm_body(x_ref, w_ref, h_ref):
    h_ref[...] = jnp.dot(x_ref[...], w_ref[...],
                         preferred_element_type=jnp.float32)


def _tc1_body(h_ref, degp_ref, g_ref, dinv_ref):
    d = degp_ref[0][:, 0:1] + degp_ref[1][:, 0:1]
    dv = lax.rsqrt(d)
    g_ref[...] = h_ref[...] * dv
    dinv_ref[...] = dv


def _tc_mid_body(acc_ref, dinv_ref, b_ref, w_ref, g_ref):
    dv = dinv_ref[...]
    h = (acc_ref[0] + acc_ref[1]) * dv + b_ref[...]
    h = jnp.maximum(h, 0.0)
    g_ref[...] = jnp.dot(h, w_ref[...], preferred_element_type=jnp.float32) * dv


def _tc_h3_body(acc_ref, dinv_ref, b_ref, h_ref, st_ref):
    i = pl.program_id(0)
    h = (acc_ref[0] + acc_ref[1]) * dinv_ref[...] + b_ref[...]
    h = jnp.maximum(h, 0.0)
    h_ref[...] = h
    s1 = jnp.sum(h, axis=0, keepdims=True)
    s2 = jnp.sum(h * h, axis=0, keepdims=True)
    st = jnp.concatenate([s1, s2], axis=0)

    @pl.when(i == 0)
    def _():
        st_ref[...] = st

    @pl.when(i != 0)
    def _():
        st_ref[...] += st


def _tc_head_body(h_ref, st_ref, gam_ref, bet_ref, wc_ref, bc_ref,
                  wr_ref, br_ref, o_ref):
    mean = st_ref[0:1] / float(N)
    var = st_ref[1:2] / float(N) - mean * mean
    xn = (h_ref[...] - mean) * lax.rsqrt(var + 1e-5) * gam_ref[...] + bet_ref[...]
    hc = jnp.dot(xn, wc_ref[...], preferred_element_type=jnp.float32) + bc_ref[...]
    hc = jnp.maximum(hc, 0.0)
    o_ref[...] = jnp.dot(hc, wr_ref[...], preferred_element_type=jnp.float32) + br_ref[...]


def _tc_mm(x, W1):
    return pl.pallas_call(
        _tc_mm_body,
        grid=(NBLK,),
        in_specs=[
            pl.BlockSpec((CH, F), lambda i: (i, 0)),
            pl.BlockSpec((F, F), lambda i: (0, 0)),
        ],
        out_specs=pl.BlockSpec((CH, F), lambda i: (i, 0)),
        out_shape=jax.ShapeDtypeStruct((NP, F), jnp.float32),
    )(x, W1)


def _tc1(h1, degp):
    return pl.pallas_call(
        _tc1_body,
        grid=(NBLK,),
        in_specs=[
            pl.BlockSpec((CH, F), lambda i: (i, 0)),
            pl.BlockSpec((2, CH, F), lambda i: (0, i, 0)),
        ],
        out_specs=[
            pl.BlockSpec((CH, F), lambda i: (i, 0)),
            pl.BlockSpec((CH, 1), lambda i: (i, 0)),
        ],
        out_shape=[
            jax.ShapeDtypeStruct((NP, F), jnp.float32),
            jax.ShapeDtypeStruct((NP, 1), jnp.float32),
        ],
    )(h1, degp)


def _tc_mid(acc, dinv, b, W):
    return pl.pallas_call(
        _tc_mid_body,
        grid=(NBLK,),
        in_specs=[
            pl.BlockSpec((2, CH, F), lambda i: (0, i, 0)),
            pl.BlockSpec((CH, 1), lambda i: (i, 0)),
            pl.BlockSpec((1, F), lambda i: (0, 0)),
            pl.BlockSpec((F, F), lambda i: (0, 0)),
        ],
        out_specs=pl.BlockSpec((CH, F), lambda i: (i, 0)),
        out_shape=jax.ShapeDtypeStruct((NP, F), jnp.float32),
    )(acc, dinv, b, W)


def _tc_h3(acc, dinv, b):
    return pl.pallas_call(
        _tc_h3_body,
        grid=(NBLK,),
        in_specs=[
            pl.BlockSpec((2, CH, F), lambda i: (0, i, 0)),
            pl.BlockSpec((CH, 1), lambda i: (i, 0)),
            pl.BlockSpec((1, F), lambda i: (0, 0)),
        ],
        out_specs=[
            pl.BlockSpec((CH, F), lambda i: (i, 0)),
            pl.BlockSpec((2, F), lambda i: (0, 0)),
        ],
        out_shape=[
            jax.ShapeDtypeStruct((N, F), jnp.float32),
            jax.ShapeDtypeStruct((2, F), jnp.float32),
        ],
    )(acc, dinv, b)


def _tc_head(h3, stats, gamma, beta, Wc, bc, Wr, br):
    return pl.pallas_call(
        _tc_head_body,
        grid=(NBLK,),
        in_specs=[
            pl.BlockSpec((CH, F), lambda i: (i, 0)),
            pl.BlockSpec((2, F), lambda i: (0, 0)),
            pl.BlockSpec((1, F), lambda i: (0, 0)),
            pl.BlockSpec((1, F), lambda i: (0, 0)),
            pl.BlockSpec((F, F), lambda i: (0, 0)),
            pl.BlockSpec((1, F), lambda i: (0, 0)),
            pl.BlockSpec((F, O), lambda i: (0, 0)),
            pl.BlockSpec((1, O), lambda i: (0, 0)),
        ],
        out_specs=pl.BlockSpec((CH, O), lambda i: (i, 0)),
        out_shape=jax.ShapeDtypeStruct((N, O), jnp.float32),
    )(h3, stats, gamma, beta, Wc, bc, Wr, br)


def kernel(x, edge_index, W1, b1, W2, b2, W3, b3, gamma, beta, Wc, bc, Wr, br):
    pad_w = EPT - EDGES_PER_TILE
    pad_src = jnp.broadcast_to(jnp.arange(pad_w, dtype=jnp.int32),
                               (NC * NS, pad_w))
    pad_dst = pad_src + SAC
    src2 = jnp.concatenate(
        [edge_index[0].reshape(NC * NS, EDGES_PER_TILE), pad_src],
        axis=1).reshape(EROWS, SE)
    dst2 = jnp.concatenate(
        [edge_index[1].reshape(NC * NS, EDGES_PER_TILE), pad_dst],
        axis=1).reshape(EROWS, SE)
    zeros = jnp.zeros((NP, F), jnp.float32)
    ones_tab = jnp.ones((NP, F), jnp.float32)

    src1p = src2.reshape(EP)
    dst1p = dst2.reshape(EP)
    h1 = _tc_mm(x, W1)
    degp = _deg_kernel(dst2, ones_tab, zeros)
    g1, dinv = _tc1(h1, degp)
    acc1 = _scatter_kernel(g1, src1p, dst1p, zeros)
    g2 = _tc_mid(acc1, dinv, b1.reshape(1, F), W2)
    acc2 = _scatter_kernel(g2, src1p, dst1p, zeros)
    g3 = _tc_mid(acc2, dinv, b2.reshape(1, F), W3)
    acc3 = _scatter_kernel(g3, src1p, dst1p, zeros)
    h3, stats = _tc_h3(acc3, dinv, b3.reshape(1, F))
    return _tc_head(h3, stats, gamma.reshape(1, F), beta.reshape(1, F),
                    Wc, bc.reshape(1, F), Wr, br.reshape(1, O))
